# baseline (speedup 1.0000x reference)
import functools

import jax
import jax.numpy as jnp
from jax import lax
from jax.experimental import pallas as pl
from jax.experimental.pallas import tpu as pltpu
from jax.experimental.pallas import tpu_sc as plsc

_L = 16
_CH = 128


@functools.lru_cache(maxsize=None)
def _build(B, D):
    info = plsc.get_sparse_core_info()
    nw = info.num_cores * info.num_subcores
    b_per_w = B // nw
    n_grp = b_per_w // _L
    n_ch = b_per_w // _CH
    n_q = D // _L
    pitch = b_per_w + 1
    mesh = plsc.VectorSubcoreMesh(core_axis_name="c", subcore_axis_name="s")

    @functools.partial(
        pl.kernel,
        mesh=mesh,
        out_type=jax.ShapeDtypeStruct((B,), jnp.float32),
        compiler_params=pltpu.CompilerParams(
            needs_layout_passes=False, use_tc_tiling_on_sc=False),
        scratch_types=[
            pltpu.VMEM((b_per_w,), jnp.int32),
            pltpu.VMEM((b_per_w,), jnp.int32),
            pltpu.VMEM((b_per_w, D), jnp.float32),
            pltpu.VMEM((b_per_w, D), jnp.float32),
            pltpu.VMEM((b_per_w,), jnp.float32),
            pltpu.VMEM((b_per_w,), jnp.float32),
            pltpu.VMEM((_L,), jnp.float32),
            pltpu.VMEM((_L * pitch,), jnp.float32),
            pltpu.VMEM((b_per_w,), jnp.float32),
            pltpu.SemaphoreType.DMA,
        ],
    )
    def k(xm, xu, me, ue, mbt, ubt, gb, out,
          idx_m, idx_u, m_rows, u_rows, mb_v, ub_v, gb_v, pT, y_v, sem):
        wid = lax.axis_index("s") * info.num_cores + lax.axis_index("c")
        base = wid * b_per_w

        pltpu.sync_copy(xm.at[pl.ds(base, b_per_w)], idx_m)
        pltpu.sync_copy(xu.at[pl.ds(base, b_per_w)], idx_u)
        pltpu.sync_copy(gb, gb_v)

        copies = []
        for c in range(n_ch):
            s = pl.ds(c * _CH, _CH)
            copies.append(pltpu.async_copy(me.at[idx_m.at[s]], m_rows.at[s], sem))
            copies.append(pltpu.async_copy(ue.at[idx_u.at[s]], u_rows.at[s], sem))
            copies.append(pltpu.async_copy(mbt.at[idx_m.at[s]], mb_v.at[s], sem))
            copies.append(pltpu.async_copy(ubt.at[idx_u.at[s]], ub_v.at[s], sem))
        for cp in copies:
            cp.wait()

        lanes = lax.iota(jnp.int32, _L) * pitch

        def pass1(b, carry):
            acc = m_rows[b, pl.ds(0, _L)] * u_rows[b, pl.ds(0, _L)]
            for q in range(1, n_q):
                acc = acc + (m_rows[b, pl.ds(q * _L, _L)]
                             * u_rows[b, pl.ds(q * _L, _L)])
            plsc.store_scatter(pT, [lanes + b], acc)
            return carry

        lax.fori_loop(0, b_per_w, pass1, 0)

        def pass2(g, carry):
            off = g * _L
            s = pT[pl.ds(off, _L)]
            for j in range(1, _L):
                s = s + pT[pl.ds(j * pitch + off, _L)]
            y = s + mb_v[pl.ds(off, _L)] + ub_v[pl.ds(off, _L)] + gb_v[...]
            y_v[pl.ds(off, _L)] = y
            return carry

        lax.fori_loop(0, n_grp, pass2, 0)

        pltpu.sync_copy(y_v, out.at[pl.ds(base, b_per_w)])

    return k


def kernel(xs, manga_emb, user_emb, manga_b, user_b, global_b):
    B = xs.shape[0]
    D = manga_emb.shape[1]
    xm = xs[:, 0]
    xu = xs[:, 1]
    gb = jnp.broadcast_to(jnp.reshape(global_b, (1,)), (_L,))
    k = _build(B, D)
    return k(xm, xu, manga_emb, user_emb, manga_b[:, 0], user_b[:, 0], gb)
